# Initial kernel scaffold; baseline (speedup 1.0000x reference)
import functools

import jax
import jax.numpy as jnp
from jax import lax
from jax.experimental import pallas as pl
from jax.experimental.pallas import tpu as pltpu
from jax.experimental.pallas import tpu_sc as plsc

D = 1024
NL = 2
NE = 4
NRQ = 512
NRKV = 512
DH = 64
H = 4
INNER = H * DH
SEQ = 4096
B = 2
NROUTE = 2 * NE
NROWS = B * NROUTE
EFF_K = min(int(NRQ * 9 / 8), SEQ)
HI = jax.lax.Precision.HIGHEST


def _k1_body(x_ref, r_ref, s_ref):
    x = x_ref[0]
    r = r_ref[...]
    s_ref[...] = lax.dot_general(r, x, (((0,), (1,)), ((), ())),
                                 precision=HI)


def _scores(x, routes):
    return pl.pallas_call(
        _k1_body,
        grid=(B, SEQ // 512),
        in_specs=[
            pl.BlockSpec((1, 512, D), lambda b, t: (b, t, 0)),
            pl.BlockSpec((D, NROUTE), lambda b, t: (0, 0)),
        ],
        out_specs=pl.BlockSpec((NROUTE, 512), lambda b, t: (b, t)),
        out_shape=jax.ShapeDtypeStruct((NROWS, SEQ), jnp.float32),
    )(x, routes)


def _cumsum_lanes(x):
    n = x.shape[-1]
    k = 1
    while k < n:
        x = x + jnp.concatenate(
            [jnp.zeros(x.shape[:-1] + (k,), x.dtype), x[..., :-k]], axis=-1)
        k *= 2
    return x


def _k2_body(s_ref, idxl_ref, idxg_ref):
    s = s_ref[...]
    logk = jnp.log(jnp.float32(EFF_K))
    b = -s
    a = jnp.zeros((NROWS, 1), jnp.float32)
    cur = 4.0
    for _ in range(20):
        sb = (s + b) / cur
        m = jnp.max(sb, axis=-1, keepdims=True)
        lse = jnp.log(jnp.sum(jnp.exp(sb - m), axis=-1, keepdims=True)) + m
        a = cur * (logk - lse)
        b = -jnp.maximum(s + a, 0.0)
        cur = max(cur * 0.7, 0.03)
    key = jnp.minimum(s + a, 0.0)
    bi = lax.bitcast_convert_type(key, jnp.int32)
    ki = jnp.where(bi >= 0, bi, bi ^ jnp.int32(0x7FFFFFFF))

    lo = jnp.full((NROWS, 1), -2139095040, jnp.int32)
    hi = jnp.zeros((NROWS, 1), jnp.int32)
    for _ in range(31):
        mid = lo + lax.shift_right_arithmetic(hi - lo + 1, 1)
        cnt = jnp.sum((ki >= mid).astype(jnp.float32), axis=-1, keepdims=True)
        pred = cnt >= float(NRQ)
        lo = jnp.where(pred, mid, lo)
        hi = jnp.where(pred, mid - 1, hi)
    T = lo

    gt = ki > T
    eq = ki == T
    c_gt = jnp.sum(gt.astype(jnp.float32), axis=-1, keepdims=True)
    need = float(NRQ) - c_gt
    eqf = eq.astype(jnp.float32)
    excl_eq = _cumsum_lanes(eqf) - eqf
    mask = jnp.logical_or(gt, jnp.logical_and(eq, excl_eq < need))
    maskf = mask.astype(jnp.float32)
    slot = _cumsum_lanes(maskf) - maskf

    jj = lax.broadcasted_iota(jnp.float32, (NRQ, SEQ), 0)
    iv = lax.broadcasted_iota(jnp.float32, (1, SEQ), 1)
    rows = []
    for r in range(NROWS):
        srow = slot[r:r + 1, :]
        mrow = maskf[r:r + 1, :]
        e = jnp.where(jnp.logical_and(srow == jj, mrow > 0.5), 1.0, 0.0)
        rows.append(lax.dot_general(iv, e, (((1,), (1,)), ((), ())),
                                    precision=HI))
    idxf = jnp.concatenate(rows, axis=0)
    idxl = idxf.astype(jnp.int32)
    roff = jnp.where(
        lax.broadcasted_iota(jnp.int32, (NROWS, 1), 0) >= NROUTE, SEQ, 0)
    idxl_ref[...] = idxl
    idxg_ref[...] = idxl + roff


def _select(s):
    return pl.pallas_call(
        _k2_body,
        in_specs=[pl.BlockSpec((NROWS, SEQ), lambda: (0, 0))],
        out_specs=[
            pl.BlockSpec((NROWS, NRQ), lambda: (0, 0)),
            pl.BlockSpec((NROWS, NRQ), lambda: (0, 0)),
        ],
        out_shape=[
            jax.ShapeDtypeStruct((NROWS, NRQ), jnp.int32),
            jax.ShapeDtypeStruct((NROWS, NRQ), jnp.int32),
        ],
    )(s)


_NW = 32
_RPW = (NROWS * NRQ) // _NW
_CH = 64
_NCH = _RPW // _CH


def _sc_gather_body(x_hbm, rot_hbm, gidx_hbm, lidx_hbm, g_hbm, grot_hbm,
                    idxg_v, idxl_v, rows_v, rrows_v, sem):
    wid = lax.axis_index("s") * 2 + lax.axis_index("c")
    pltpu.sync_copy(gidx_hbm.at[wid], idxg_v)
    pltpu.sync_copy(lidx_hbm.at[wid], idxl_v)
    base = wid * _RPW
    for c in range(_NCH):
        pltpu.async_copy(x_hbm.at[idxg_v.at[c]], rows_v, sem).wait()
        pltpu.sync_copy(rows_v, g_hbm.at[pl.ds(base + c * _CH, _CH)])
        pltpu.async_copy(rot_hbm.at[idxl_v.at[c]], rrows_v, sem).wait()
        pltpu.sync_copy(rrows_v, grot_hbm.at[pl.ds(base + c * _CH, _CH)])


def _sc_gather(x2d, rot, idxg, idxl):
    mesh = plsc.VectorSubcoreMesh(core_axis_name="c", subcore_axis_name="s")
    fn = functools.partial(
        pl.kernel, mesh=mesh,
        out_type=[
            jax.ShapeDtypeStruct((NROWS * NRQ, D), jnp.float32),
            jax.ShapeDtypeStruct((NROWS * NRQ, DH), jnp.float32),
        ],
        scratch_types=[
            pltpu.VMEM((_NCH, _CH), jnp.int32),
            pltpu.VMEM((_NCH, _CH), jnp.int32),
            pltpu.VMEM((_CH, D), jnp.float32),
            pltpu.VMEM((_CH, DH), jnp.float32),
            pltpu.SemaphoreType.DMA,
        ],
    )(_sc_gather_body)
    return fn(x2d, rot,
              idxg.reshape(_NW, _NCH, _CH), idxl.reshape(_NW, _NCH, _CH))


def _k4_body(gq_ref, gkv_ref, rq_ref, rkv_ref, lng_ref, lnb_ref,
             wq_ref, wkv_ref, wo_ref, nkv_ref, ntok_ref, out_ref):
    gq = gq_ref[0, 0]
    gkv = gkv_ref[0, 0]
    rq = rq_ref[0, 0]
    rkv = rkv_ref[0, 0]
    g = lng_ref[...]
    bb = lnb_ref[...]

    def ln(t):
        mu = jnp.mean(t, axis=-1, keepdims=True)
        var = jnp.mean((t - mu) ** 2, axis=-1, keepdims=True)
        return (t - mu) / jnp.sqrt(var + 1e-5) * g + bb

    xq = ln(gq)
    ctx = ln(gkv)
    q = lax.dot_general(xq, wq_ref[0], (((1,), (0,)), ((), ())),
                        precision=HI)
    kv = lax.dot_general(ctx, wkv_ref[0], (((1,), (0,)), ((), ())),
                         precision=HI)
    k = kv[:, :INNER]
    v = kv[:, INNER:]

    cq, sq = jnp.cos(rq), jnp.sin(rq)
    ck, sk = jnp.cos(rkv), jnp.sin(rkv)
    cq4 = jnp.concatenate([cq] * H, axis=1)
    sq4 = jnp.concatenate([sq] * H, axis=1)
    ck4 = jnp.concatenate([ck] * H, axis=1)
    sk4 = jnp.concatenate([sk] * H, axis=1)

    def rot_half(t):
        parts = []
        for h in range(H):
            t1 = t[:, h * DH:h * DH + DH // 2]
            t2 = t[:, h * DH + DH // 2:(h + 1) * DH]
            parts.append(jnp.concatenate([-t2, t1], axis=1))
        return jnp.concatenate(parts, axis=1)

    q = q * cq4 + rot_half(q) * sq4
    k = k * ck4 + rot_half(k) * sk4

    nkv = nkv_ref[0]
    scale = DH ** -0.5
    outs = []
    for h in range(H):
        qh = q[:, h * DH:(h + 1) * DH]
        kh = k[:, h * DH:(h + 1) * DH]
        vh = v[:, h * DH:(h + 1) * DH]
        sim = lax.dot_general(qh, kh, (((1,), (1,)), ((), ())),
                              precision=HI) * scale
        nk = nkv[0, h:h + 1, :]
        nv = nkv[1, h:h + 1, :]
        lnull = lax.dot_general(qh, nk, (((1,), (1,)), ((), ())),
                                precision=HI) * scale
        m = jnp.maximum(jnp.max(sim, axis=-1, keepdims=True), lnull)
        p = jnp.exp(sim - m)
        pn = jnp.exp(lnull - m)
        den = jnp.sum(p, axis=-1, keepdims=True) + pn
        oh = (lax.dot_general(p, vh, (((1,), (0,)), ((), ())), precision=HI)
              + pn * nv) / den
        outs.append(oh)
    o = jnp.concatenate(outs, axis=1)
    ao = lax.dot_general(o, wo_ref[0], (((1,), (0,)), ((), ())),
                         precision=HI)
    out_ref[0, 0] = (ao - ntok_ref[...]) * (1.0 / NE)


def _experts(g4, grot4, lng, lnb, wq, wkv, wo, nkv, ntok):
    return pl.pallas_call(
        _k4_body,
        grid=(NE, B),
        in_specs=[
            pl.BlockSpec((1, 1, NRQ, D), lambda e, b: (b, e, 0, 0)),
            pl.BlockSpec((1, 1, NRKV, D), lambda e, b: (b, e + NE, 0, 0)),
            pl.BlockSpec((1, 1, NRQ, DH), lambda e, b: (b, e, 0, 0)),
            pl.BlockSpec((1, 1, NRKV, DH), lambda e, b: (b, e + NE, 0, 0)),
            pl.BlockSpec((1, D), lambda e, b: (e, 0)),
            pl.BlockSpec((1, D), lambda e, b: (e, 0)),
            pl.BlockSpec((1, D, INNER), lambda e, b: (e, 0, 0)),
            pl.BlockSpec((1, D, 2 * INNER), lambda e, b: (e, 0, 0)),
            pl.BlockSpec((1, INNER, D), lambda e, b: (e, 0, 0)),
            pl.BlockSpec((1, 2, H, DH), lambda e, b: (e, 0, 0, 0)),
            pl.BlockSpec((1, D), lambda e, b: (e, 0)),
        ],
        out_specs=pl.BlockSpec((1, 1, NRQ, D), lambda e, b: (e, b, 0, 0)),
        out_shape=jax.ShapeDtypeStruct((NE, B, NRQ, D), jnp.float32),
    )(g4, grot4, lng, lnb, wq, wkv, wo, nkv, ntok)


def _k5_body(x_ref, d_ref, idx_ref, mn_ref, ffg_ref, ffb_ref,
             w1_ref, b1_ref, w2_ref, b2_ref, out_ref, *, tile):
    t = pl.program_id(1)
    xt = x_ref[0]
    rowid = (lax.broadcasted_iota(jnp.int32, (tile, 1), 0)
             + t * tile)
    acc = xt + mn_ref[...]
    for e in range(NE):
        ide = idx_ref[e:e + 1, :]
        me = jnp.where(ide == rowid, 1.0, 0.0)
        acc = acc + lax.dot_general(me, d_ref[e, 0],
                                    (((1,), (0,)), ((), ())), precision=HI)
    mu = jnp.mean(acc, axis=-1, keepdims=True)
    var = jnp.mean((acc - mu) ** 2, axis=-1, keepdims=True)
    h = (acc - mu) / jnp.sqrt(var + 1e-5) * ffg_ref[...] + ffb_ref[...]
    h = lax.dot_general(h, w1_ref[...], (((1,), (0,)), ((), ())),
                        precision=HI) + b1_ref[...]
    h = jax.nn.gelu(h, approximate=False)
    h = lax.dot_general(h, w2_ref[...], (((1,), (0,)), ((), ())),
                        precision=HI) + b2_ref[...]
    out_ref[0] = h + acc


def _combine_ff(x3, deltas, idxl, mn, ffg, ffb, w1, b1, w2, b2):
    tile = 512
    body = functools.partial(_k5_body, tile=tile)
    return pl.pallas_call(
        body,
        grid=(B, SEQ // tile),
        in_specs=[
            pl.BlockSpec((1, tile, D), lambda b, t: (b, t, 0)),
            pl.BlockSpec((NE, 1, NRQ, D), lambda b, t: (0, b, 0, 0)),
            pl.BlockSpec((NROUTE, NRQ), lambda b, t: (b, 0)),
            pl.BlockSpec((1, D), lambda b, t: (0, 0)),
            pl.BlockSpec((1, D), lambda b, t: (0, 0)),
            pl.BlockSpec((1, D), lambda b, t: (0, 0)),
            pl.BlockSpec((D, D), lambda b, t: (0, 0)),
            pl.BlockSpec((1, D), lambda b, t: (0, 0)),
            pl.BlockSpec((D, D), lambda b, t: (0, 0)),
            pl.BlockSpec((1, D), lambda b, t: (0, 0)),
        ],
        out_specs=pl.BlockSpec((1, tile, D), lambda b, t: (b, t, 0)),
        out_shape=jax.ShapeDtypeStruct((B, SEQ, D), jnp.float32),
    )(x3, deltas, idxl, mn, ffg, ffb, w1, b1, w2, b2)


def _k6_body(x_ref, g_ref, b_ref, o_ref):
    xt = x_ref[...]
    mu = jnp.mean(xt, axis=-1, keepdims=True)
    var = jnp.mean((xt - mu) ** 2, axis=-1, keepdims=True)
    o_ref[...] = (xt - mu) / jnp.sqrt(var + 1e-5) * g_ref[...] + b_ref[...]


def _final_ln(x2d, g, b):
    return pl.pallas_call(
        _k6_body,
        grid=(B * SEQ // 512,),
        in_specs=[
            pl.BlockSpec((512, D), lambda t: (t, 0)),
            pl.BlockSpec((1, D), lambda t: (0, 0)),
            pl.BlockSpec((1, D), lambda t: (0, 0)),
        ],
        out_specs=pl.BlockSpec((512, D), lambda t: (t, 0)),
        out_shape=jax.ShapeDtypeStruct((B * SEQ, D), jnp.float32),
    )(x2d, g, b)


def kernel(x, rotary_emb, params):
    xc = x
    for layer in params['layers']:
        ex = layer['experts']
        routes = jnp.stack([e['q_route'] for e in ex]
                           + [e['kv_route'] for e in ex], axis=1)
        s = _scores(xc, routes)
        idxl, idxg = _select(s)
        g, grot = _sc_gather(xc.reshape(B * SEQ, D), rotary_emb,
                             idxg.reshape(-1), idxl.reshape(-1))
        g4 = g.reshape(B, NROUTE, NRQ, D)
        grot4 = grot.reshape(B, NROUTE, NRQ, DH)
        lng = jnp.stack([e['ln_g'] for e in ex])
        lnb = jnp.stack([e['ln_b'] for e in ex])
        wq = jnp.stack([e['Wq'] for e in ex])
        wkv = jnp.stack([e['Wkv'] for e in ex])
        wo = jnp.stack([e['Wo'] for e in ex])
        nkv = jnp.stack([e['null_kv'] for e in ex])
        ntok = jnp.stack([e['null_tokens'][0, 0] for e in ex])
        deltas = _experts(g4, grot4, lng, lnb, wq, wkv, wo, nkv, ntok)
        mn = jnp.mean(ntok, axis=0, keepdims=True)
        ff = layer['ff']
        xc = _combine_ff(
            xc, deltas, idxl, mn,
            ff['ln_g'][None, :], ff['ln_b'][None, :],
            ff['W1'], ff['b1'][None, :], ff['W2'], ff['b2'][None, :])
    out = _final_ln(xc.reshape(B * SEQ, D),
                    params['out_ln_g'][None, :], params['out_ln_b'][None, :])
    return out.reshape(B, SEQ, D)


# speedup vs baseline: 2.0494x; 2.0494x over previous
import functools

import jax
import jax.numpy as jnp
from jax import lax
from jax.experimental import pallas as pl
from jax.experimental.pallas import tpu as pltpu
from jax.experimental.pallas import tpu_sc as plsc

D = 1024
NL = 2
NE = 4
NRQ = 512
NRKV = 512
DH = 64
H = 4
INNER = H * DH
SEQ = 4096
B = 2
NROUTE = 2 * NE
NROWS = B * NROUTE
EFF_K = min(int(NRQ * 9 / 8), SEQ)
HI = jax.lax.Precision.HIGHEST


def _k1_body(x_ref, r_ref, s_ref):
    x = x_ref[0]
    r = r_ref[...]
    s_ref[...] = lax.dot_general(r, x, (((0,), (1,)), ((), ())),
                                 precision=jax.lax.Precision.DEFAULT)


def _scores(x, routes):
    return pl.pallas_call(
        _k1_body,
        grid=(B, SEQ // 512),
        in_specs=[
            pl.BlockSpec((1, 512, D), lambda b, t: (b, t, 0)),
            pl.BlockSpec((D, NROUTE), lambda b, t: (0, 0)),
        ],
        out_specs=pl.BlockSpec((NROUTE, 512), lambda b, t: (b, t)),
        out_shape=jax.ShapeDtypeStruct((NROWS, SEQ), jnp.float32),
    )(x, routes)


def _cumsum_lanes(x):
    n = x.shape[-1]
    k = 1
    while k < n:
        x = x + jnp.concatenate(
            [jnp.zeros(x.shape[:-1] + (k,), x.dtype), x[..., :-k]], axis=-1)
        k *= 2
    return x


def _k2_body(s_ref, idxl_ref, idxg_ref):
    s = s_ref[...]
    logk = jnp.log(jnp.float32(EFF_K))
    b = -s
    a = jnp.zeros((NROWS, 1), jnp.float32)
    cur = 4.0
    for _ in range(20):
        sb = (s + b) / cur
        m = jnp.max(sb, axis=-1, keepdims=True)
        lse = jnp.log(jnp.sum(jnp.exp(sb - m), axis=-1, keepdims=True)) + m
        a = cur * (logk - lse)
        b = -jnp.maximum(s + a, 0.0)
        cur = max(cur * 0.7, 0.03)
    key = jnp.minimum(s + a, 0.0)
    bi = lax.bitcast_convert_type(key, jnp.int32)
    ki = jnp.where(bi >= 0, bi, bi ^ jnp.int32(0x7FFFFFFF))

    lo = jnp.full((NROWS, 1), -2139095040, jnp.int32)
    hi = jnp.zeros((NROWS, 1), jnp.int32)
    for _ in range(31):
        mid = lo + lax.shift_right_arithmetic(hi - lo + 1, 1)
        cnt = jnp.sum((ki >= mid).astype(jnp.float32), axis=-1, keepdims=True)
        pred = cnt >= float(NRQ)
        lo = jnp.where(pred, mid, lo)
        hi = jnp.where(pred, hi, mid - 1)
    T = lo

    gt = ki > T
    eq = ki == T
    c_gt = jnp.sum(gt.astype(jnp.float32), axis=-1, keepdims=True)
    need = float(NRQ) - c_gt
    eqf = eq.astype(jnp.float32)
    excl_eq = _cumsum_lanes(eqf) - eqf
    mask = jnp.logical_or(gt, jnp.logical_and(eq, excl_eq < need))
    maskf = mask.astype(jnp.float32)
    slot = _cumsum_lanes(maskf) - maskf

    jj = lax.broadcasted_iota(jnp.int32, (NRQ, SEQ), 0).astype(jnp.float32)
    iv = lax.broadcasted_iota(jnp.int32, (1, SEQ), 1).astype(jnp.float32)
    rows = []
    for r in range(NROWS):
        srow = slot[r:r + 1, :]
        mrow = maskf[r:r + 1, :]
        e = jnp.where(jnp.logical_and(srow == jj, mrow > 0.5), 1.0, 0.0)
        rows.append(lax.dot_general(iv, e, (((1,), (1,)), ((), ())),
                                    precision=HI))
    idxf = jnp.concatenate(rows, axis=0)
    idxl = idxf.astype(jnp.int32)
    roff = jnp.where(
        lax.broadcasted_iota(jnp.int32, (NROWS, 1), 0) >= NROUTE, SEQ, 0)
    idxl_ref[...] = idxl
    idxg_ref[...] = idxl + roff


def _select(s):
    return pl.pallas_call(
        _k2_body,
        in_specs=[pl.BlockSpec((NROWS, SEQ), lambda: (0, 0))],
        out_specs=[
            pl.BlockSpec((NROWS, NRQ), lambda: (0, 0)),
            pl.BlockSpec((NROWS, NRQ), lambda: (0, 0)),
        ],
        out_shape=[
            jax.ShapeDtypeStruct((NROWS, NRQ), jnp.int32),
            jax.ShapeDtypeStruct((NROWS, NRQ), jnp.int32),
        ],
    )(s)


_NW = 32
_RPW = (NROWS * NRQ) // _NW
_CH = 64
_NCH = _RPW // _CH


def _sc_gather_body(x_hbm, rot_hbm, gidx_hbm, lidx_hbm, g_hbm, grot_hbm,
                    idxg_v, idxl_v, rows_v, rrows_v, sem):
    wid = lax.axis_index("s") * 2 + lax.axis_index("c")
    pltpu.sync_copy(gidx_hbm.at[wid], idxg_v)
    pltpu.sync_copy(lidx_hbm.at[wid], idxl_v)
    base = wid * _RPW
    for c in range(_NCH):
        pltpu.async_copy(x_hbm.at[idxg_v.at[c]], rows_v, sem).wait()
        pltpu.sync_copy(rows_v, g_hbm.at[pl.ds(base + c * _CH, _CH)])
        pltpu.async_copy(rot_hbm.at[idxl_v.at[c]], rrows_v, sem).wait()
        pltpu.sync_copy(rrows_v, grot_hbm.at[pl.ds(base + c * _CH, _CH)])


def _sc_gather(x2d, rot, idxg, idxl):
    mesh = plsc.VectorSubcoreMesh(core_axis_name="c", subcore_axis_name="s")
    fn = functools.partial(
        pl.kernel, mesh=mesh,
        out_type=[
            jax.ShapeDtypeStruct((NROWS * NRQ, D), jnp.float32),
            jax.ShapeDtypeStruct((NROWS * NRQ, 2 * DH), jnp.float32),
        ],
        scratch_types=[
            pltpu.VMEM((_NCH, _CH), jnp.int32),
            pltpu.VMEM((_NCH, _CH), jnp.int32),
            pltpu.VMEM((_CH, D), jnp.float32),
            pltpu.VMEM((_CH, 2 * DH), jnp.float32),
            pltpu.SemaphoreType.DMA,
        ],
    )(_sc_gather_body)
    return fn(x2d, rot,
              idxg.reshape(_NW, _NCH, _CH), idxl.reshape(_NW, _NCH, _CH))


def _k4_body(gq_ref, gkv_ref, rq_ref, rkv_ref, lng_ref, lnb_ref,
             wq_ref, wkv_ref, wo_ref, nkv_ref, ntok_ref, out_ref):
    gq = gq_ref[0, 0]
    gkv = gkv_ref[0, 0]
    rq = rq_ref[0, 0, :, :DH]
    rkv = rkv_ref[0, 0, :, :DH]
    g = lng_ref[0]
    bb = lnb_ref[0]

    def ln(t):
        mu = jnp.mean(t, axis=-1, keepdims=True)
        var = jnp.mean((t - mu) ** 2, axis=-1, keepdims=True)
        return (t - mu) / jnp.sqrt(var + 1e-5) * g + bb

    xq = ln(gq)
    ctx = ln(gkv)
    q = lax.dot_general(xq, wq_ref[0], (((1,), (0,)), ((), ())))
    kv = lax.dot_general(ctx, wkv_ref[0], (((1,), (0,)), ((), ())))
    k = kv[:, :INNER]
    v = kv[:, INNER:]

    cq, sq = jnp.cos(rq), jnp.sin(rq)
    ck, sk = jnp.cos(rkv), jnp.sin(rkv)
    cq4 = jnp.concatenate([cq] * H, axis=1)
    sq4 = jnp.concatenate([sq] * H, axis=1)
    ck4 = jnp.concatenate([ck] * H, axis=1)
    sk4 = jnp.concatenate([sk] * H, axis=1)

    def rot_half(t):
        parts = []
        for h in range(H):
            t1 = t[:, h * DH:h * DH + DH // 2]
            t2 = t[:, h * DH + DH // 2:(h + 1) * DH]
            parts.append(jnp.concatenate([-t2, t1], axis=1))
        return jnp.concatenate(parts, axis=1)

    q = q * cq4 + rot_half(q) * sq4
    k = k * ck4 + rot_half(k) * sk4

    nkv = nkv_ref[0]
    scale = DH ** -0.5
    outs = []
    for h in range(H):
        qh = q[:, h * DH:(h + 1) * DH]
        kh = k[:, h * DH:(h + 1) * DH]
        vh = v[:, h * DH:(h + 1) * DH]
        sim = lax.dot_general(qh, kh, (((1,), (1,)), ((), ()))) * scale
        nk = nkv[0, h:h + 1, :]
        nv = nkv[1, h:h + 1, :]
        lnull = lax.dot_general(qh, nk, (((1,), (1,)), ((), ()))) * scale
        m = jnp.maximum(jnp.max(sim, axis=-1, keepdims=True), lnull)
        p = jnp.exp(sim - m)
        pn = jnp.exp(lnull - m)
        den = jnp.sum(p, axis=-1, keepdims=True) + pn
        oh = (lax.dot_general(p, vh, (((1,), (0,)), ((), ())))
              + pn * nv) / den
        outs.append(oh)
    o = jnp.concatenate(outs, axis=1)
    ao = lax.dot_general(o, wo_ref[0], (((1,), (0,)), ((), ())))
    out_ref[0, 0] = (ao - ntok_ref[0]) * (1.0 / NE)


def _experts(g4, grot4, lng, lnb, wq, wkv, wo, nkv, ntok):
    return pl.pallas_call(
        _k4_body,
        grid=(NE, B),
        in_specs=[
            pl.BlockSpec((1, 1, NRQ, D), lambda e, b: (b, e, 0, 0)),
            pl.BlockSpec((1, 1, NRKV, D), lambda e, b: (b, e + NE, 0, 0)),
            pl.BlockSpec((1, 1, NRQ, 2 * DH), lambda e, b: (b, e, 0, 0)),
            pl.BlockSpec((1, 1, NRKV, 2 * DH), lambda e, b: (b, e + NE, 0, 0)),
            pl.BlockSpec((1, 1, D), lambda e, b: (e, 0, 0)),
            pl.BlockSpec((1, 1, D), lambda e, b: (e, 0, 0)),
            pl.BlockSpec((1, D, INNER), lambda e, b: (e, 0, 0)),
            pl.BlockSpec((1, D, 2 * INNER), lambda e, b: (e, 0, 0)),
            pl.BlockSpec((1, INNER, D), lambda e, b: (e, 0, 0)),
            pl.BlockSpec((1, 2, H, DH), lambda e, b: (e, 0, 0, 0)),
            pl.BlockSpec((1, 1, D), lambda e, b: (e, 0, 0)),
        ],
        out_specs=pl.BlockSpec((1, 1, NRQ, D), lambda e, b: (e, b, 0, 0)),
        out_shape=jax.ShapeDtypeStruct((NE, B, NRQ, D), jnp.float32),
    )(g4, g4, grot4, grot4, lng, lnb, wq, wkv, wo, nkv, ntok)


def _k5_body(x_ref, d_ref, idx_ref, mn_ref, ffg_ref, ffb_ref,
             w1_ref, b1_ref, w2_ref, b2_ref, out_ref, *, tile):
    t = pl.program_id(1)
    xt = x_ref[0]
    rowid = (lax.broadcasted_iota(jnp.int32, (tile, 1), 0)
             + t * tile)
    acc = xt + mn_ref[...]
    for e in range(NE):
        ide = idx_ref[e:e + 1, :]
        me = jnp.where(ide == rowid, 1.0, 0.0)
        acc = acc + lax.dot_general(me, d_ref[e, 0],
                                    (((1,), (0,)), ((), ())), precision=HI)
    mu = jnp.mean(acc, axis=-1, keepdims=True)
    var = jnp.mean((acc - mu) ** 2, axis=-1, keepdims=True)
    h = (acc - mu) / jnp.sqrt(var + 1e-5) * ffg_ref[...] + ffb_ref[...]
    h = lax.dot_general(h, w1_ref[...], (((1,), (0,)), ((), ()))) + b1_ref[...]
    h = 0.5 * h * (1.0 + lax.erf(h * (2.0 ** -0.5)))
    h = lax.dot_general(h, w2_ref[...], (((1,), (0,)), ((), ()))) + b2_ref[...]
    out_ref[0] = h + acc


def _combine_ff(x3, deltas, idxl, mn, ffg, ffb, w1, b1, w2, b2):
    tile = 512
    body = functools.partial(_k5_body, tile=tile)
    return pl.pallas_call(
        body,
        grid=(B, SEQ // tile),
        in_specs=[
            pl.BlockSpec((1, tile, D), lambda b, t: (b, t, 0)),
            pl.BlockSpec((NE, 1, NRQ, D), lambda b, t: (0, b, 0, 0)),
            pl.BlockSpec((NROUTE, NRQ), lambda b, t: (b, 0)),
            pl.BlockSpec((1, D), lambda b, t: (0, 0)),
            pl.BlockSpec((1, D), lambda b, t: (0, 0)),
            pl.BlockSpec((1, D), lambda b, t: (0, 0)),
            pl.BlockSpec((D, D), lambda b, t: (0, 0)),
            pl.BlockSpec((1, D), lambda b, t: (0, 0)),
            pl.BlockSpec((D, D), lambda b, t: (0, 0)),
            pl.BlockSpec((1, D), lambda b, t: (0, 0)),
        ],
        out_specs=pl.BlockSpec((1, tile, D), lambda b, t: (b, t, 0)),
        out_shape=jax.ShapeDtypeStruct((B, SEQ, D), jnp.float32),
    )(x3, deltas, idxl, mn, ffg, ffb, w1, b1, w2, b2)


def _k6_body(x_ref, g_ref, b_ref, o_ref):
    xt = x_ref[...]
    mu = jnp.mean(xt, axis=-1, keepdims=True)
    var = jnp.mean((xt - mu) ** 2, axis=-1, keepdims=True)
    o_ref[...] = (xt - mu) / jnp.sqrt(var + 1e-5) * g_ref[...] + b_ref[...]


def _final_ln(x2d, g, b):
    return pl.pallas_call(
        _k6_body,
        grid=(B * SEQ // 512,),
        in_specs=[
            pl.BlockSpec((512, D), lambda t: (t, 0)),
            pl.BlockSpec((1, D), lambda t: (0, 0)),
            pl.BlockSpec((1, D), lambda t: (0, 0)),
        ],
        out_specs=pl.BlockSpec((512, D), lambda t: (t, 0)),
        out_shape=jax.ShapeDtypeStruct((B * SEQ, D), jnp.float32),
    )(x2d, g, b)


def kernel(x, rotary_emb, params):
    xc = x
    for layer in params['layers']:
        ex = layer['experts']
        routes = jnp.stack([e['q_route'] for e in ex]
                           + [e['kv_route'] for e in ex], axis=1)
        s = _scores(xc, routes)
        idxl, idxg = _select(s)
        rot128 = jnp.concatenate(
            [rotary_emb, jnp.zeros((SEQ, DH), jnp.float32)], axis=1)
        g, grot = _sc_gather(xc.reshape(B * SEQ, D), rot128,
                             idxg.reshape(-1), idxl.reshape(-1))
        g4 = g.reshape(B, NROUTE, NRQ, D)
        grot4 = grot.reshape(B, NROUTE, NRQ, 2 * DH)
        lng = jnp.stack([e['ln_g'] for e in ex])[:, None, :]
        lnb = jnp.stack([e['ln_b'] for e in ex])[:, None, :]
        wq = jnp.stack([e['Wq'] for e in ex])
        wkv = jnp.stack([e['Wkv'] for e in ex])
        wo = jnp.stack([e['Wo'] for e in ex])
        nkv = jnp.stack([e['null_kv'] for e in ex])
        ntok = jnp.stack([e['null_tokens'][0, 0] for e in ex])[:, None, :]
        deltas = _experts(g4, grot4, lng, lnb, wq, wkv, wo, nkv, ntok)
        mn = jnp.mean(ntok[:, 0, :], axis=0, keepdims=True)
        ff = layer['ff']
        xc = _combine_ff(
            xc, deltas, idxl, mn,
            ff['ln_g'][None, :], ff['ln_b'][None, :],
            ff['W1'], ff['b1'][None, :], ff['W2'], ff['b2'][None, :])
    out = _final_ln(xc.reshape(B * SEQ, D),
                    params['out_ln_g'][None, :], params['out_ln_b'][None, :])
    return out.reshape(B, SEQ, D)


# speedup vs baseline: 2.0514x; 1.0010x over previous
import functools

import jax
import jax.numpy as jnp
from jax import lax
from jax.experimental import pallas as pl
from jax.experimental.pallas import tpu as pltpu
from jax.experimental.pallas import tpu_sc as plsc

D = 1024
NL = 2
NE = 4
NRQ = 512
NRKV = 512
DH = 64
H = 4
INNER = H * DH
SEQ = 4096
B = 2
NROUTE = 2 * NE
NROWS = B * NROUTE
EFF_K = min(int(NRQ * 9 / 8), SEQ)
HI = jax.lax.Precision.HIGHEST


def _k1_body(x_ref, r_ref, s_ref):
    x = x_ref[0]
    r = r_ref[...]
    s_ref[...] = lax.dot_general(r, x, (((0,), (1,)), ((), ())),
                                 precision=jax.lax.Precision.DEFAULT)


def _scores(x, routes):
    return pl.pallas_call(
        _k1_body,
        grid=(B, SEQ // 512),
        in_specs=[
            pl.BlockSpec((1, 512, D), lambda b, t: (b, t, 0)),
            pl.BlockSpec((D, NROUTE), lambda b, t: (0, 0)),
        ],
        out_specs=pl.BlockSpec((NROUTE, 512), lambda b, t: (b, t)),
        out_shape=jax.ShapeDtypeStruct((NROWS, SEQ), jnp.float32),
    )(x, routes)


def _cumsum_lanes(x):
    n = x.shape[-1]
    k = 1
    while k < n:
        x = x + jnp.concatenate(
            [jnp.zeros(x.shape[:-1] + (k,), x.dtype), x[..., :-k]], axis=-1)
        k *= 2
    return x


def _k2_body(s_ref, idxl_ref, idxg_ref):
    s = s_ref[...]
    logk = jnp.log(jnp.float32(EFF_K))
    b = -s
    a = jnp.zeros((NROWS, 1), jnp.float32)
    cur = 4.0
    for _ in range(20):
        sb = (s + b) / cur
        m = jnp.max(sb, axis=-1, keepdims=True)
        lse = jnp.log(jnp.sum(jnp.exp(sb - m), axis=-1, keepdims=True)) + m
        a = cur * (logk - lse)
        b = -jnp.maximum(s + a, 0.0)
        cur = max(cur * 0.7, 0.03)
    key = jnp.minimum(s + a, 0.0)
    scr = jnp.exp(key / cur)
    ki = lax.bitcast_convert_type(scr, jnp.int32)

    lo = jnp.zeros((NROWS, 1), jnp.int32)
    hi = jnp.full((NROWS, 1), 1065353216, jnp.int32)
    for _ in range(31):
        mid = lo + lax.shift_right_arithmetic(hi - lo + 1, 1)
        cnt = jnp.sum((ki >= mid).astype(jnp.float32), axis=-1, keepdims=True)
        pred = cnt >= float(NRQ)
        lo = jnp.where(pred, mid, lo)
        hi = jnp.where(pred, hi, mid - 1)
    T = lo

    gt = ki > T
    eq = ki == T
    c_gt = jnp.sum(gt.astype(jnp.float32), axis=-1, keepdims=True)
    need = float(NRQ) - c_gt
    eqf = eq.astype(jnp.float32)
    excl_eq = _cumsum_lanes(eqf) - eqf
    mask = jnp.logical_or(gt, jnp.logical_and(eq, excl_eq < need))
    maskf = mask.astype(jnp.float32)
    slot = _cumsum_lanes(maskf) - maskf

    jj = lax.broadcasted_iota(jnp.int32, (NRQ, SEQ), 0).astype(jnp.float32)
    iv = lax.broadcasted_iota(jnp.int32, (1, SEQ), 1).astype(jnp.float32)
    rows = []
    for r in range(NROWS):
        srow = slot[r:r + 1, :]
        mrow = maskf[r:r + 1, :]
        e = jnp.where(jnp.logical_and(srow == jj, mrow > 0.5), 1.0, 0.0)
        rows.append(lax.dot_general(iv, e, (((1,), (1,)), ((), ())),
                                    precision=HI))
    idxf = jnp.concatenate(rows, axis=0)
    idxl = idxf.astype(jnp.int32)
    roff = jnp.where(
        lax.broadcasted_iota(jnp.int32, (NROWS, 1), 0) >= NROUTE, SEQ, 0)
    idxl_ref[...] = idxl
    idxg_ref[...] = idxl + roff


def _select(s):
    return pl.pallas_call(
        _k2_body,
        in_specs=[pl.BlockSpec((NROWS, SEQ), lambda: (0, 0))],
        out_specs=[
            pl.BlockSpec((NROWS, NRQ), lambda: (0, 0)),
            pl.BlockSpec((NROWS, NRQ), lambda: (0, 0)),
        ],
        out_shape=[
            jax.ShapeDtypeStruct((NROWS, NRQ), jnp.int32),
            jax.ShapeDtypeStruct((NROWS, NRQ), jnp.int32),
        ],
    )(s)


_NW = 32
_RPW = (NROWS * NRQ) // _NW
_CH = 64
_NCH = _RPW // _CH


def _sc_gather_body(x_hbm, rot_hbm, gidx_hbm, lidx_hbm, g_hbm, grot_hbm,
                    idxg_v, idxl_v, rows_v, rrows_v, sem):
    wid = lax.axis_index("s") * 2 + lax.axis_index("c")
    pltpu.sync_copy(gidx_hbm.at[wid], idxg_v)
    pltpu.sync_copy(lidx_hbm.at[wid], idxl_v)
    base = wid * _RPW
    for c in range(_NCH):
        pltpu.async_copy(x_hbm.at[idxg_v.at[c]], rows_v, sem).wait()
        pltpu.sync_copy(rows_v, g_hbm.at[pl.ds(base + c * _CH, _CH)])
        pltpu.async_copy(rot_hbm.at[idxl_v.at[c]], rrows_v, sem).wait()
        pltpu.sync_copy(rrows_v, grot_hbm.at[pl.ds(base + c * _CH, _CH)])


def _sc_gather(x2d, rot, idxg, idxl):
    mesh = plsc.VectorSubcoreMesh(core_axis_name="c", subcore_axis_name="s")
    fn = functools.partial(
        pl.kernel, mesh=mesh,
        out_type=[
            jax.ShapeDtypeStruct((NROWS * NRQ, D), jnp.float32),
            jax.ShapeDtypeStruct((NROWS * NRQ, 2 * DH), jnp.float32),
        ],
        scratch_types=[
            pltpu.VMEM((_NCH, _CH), jnp.int32),
            pltpu.VMEM((_NCH, _CH), jnp.int32),
            pltpu.VMEM((_CH, D), jnp.float32),
            pltpu.VMEM((_CH, 2 * DH), jnp.float32),
            pltpu.SemaphoreType.DMA,
        ],
    )(_sc_gather_body)
    return fn(x2d, rot,
              idxg.reshape(_NW, _NCH, _CH), idxl.reshape(_NW, _NCH, _CH))


def _k4_body(gq_ref, gkv_ref, rq_ref, rkv_ref, lng_ref, lnb_ref,
             wq_ref, wkv_ref, wo_ref, nkv_ref, ntok_ref, out_ref):
    gq = gq_ref[0, 0]
    gkv = gkv_ref[0, 0]
    rq = rq_ref[0, 0, :, :DH]
    rkv = rkv_ref[0, 0, :, :DH]
    g = lng_ref[0]
    bb = lnb_ref[0]

    def ln(t):
        mu = jnp.mean(t, axis=-1, keepdims=True)
        var = jnp.mean((t - mu) ** 2, axis=-1, keepdims=True)
        return (t - mu) / jnp.sqrt(var + 1e-5) * g + bb

    xq = ln(gq)
    ctx = ln(gkv)
    q = lax.dot_general(xq, wq_ref[0], (((1,), (0,)), ((), ())))
    kv = lax.dot_general(ctx, wkv_ref[0], (((1,), (0,)), ((), ())))
    k = kv[:, :INNER]
    v = kv[:, INNER:]

    cq, sq = jnp.cos(rq), jnp.sin(rq)
    ck, sk = jnp.cos(rkv), jnp.sin(rkv)
    cq4 = jnp.concatenate([cq] * H, axis=1)
    sq4 = jnp.concatenate([sq] * H, axis=1)
    ck4 = jnp.concatenate([ck] * H, axis=1)
    sk4 = jnp.concatenate([sk] * H, axis=1)

    def rot_half(t):
        parts = []
        for h in range(H):
            t1 = t[:, h * DH:h * DH + DH // 2]
            t2 = t[:, h * DH + DH // 2:(h + 1) * DH]
            parts.append(jnp.concatenate([-t2, t1], axis=1))
        return jnp.concatenate(parts, axis=1)

    q = q * cq4 + rot_half(q) * sq4
    k = k * ck4 + rot_half(k) * sk4

    nkv = nkv_ref[0]
    scale = DH ** -0.5
    outs = []
    for h in range(H):
        qh = q[:, h * DH:(h + 1) * DH]
        kh = k[:, h * DH:(h + 1) * DH]
        vh = v[:, h * DH:(h + 1) * DH]
        sim = lax.dot_general(qh, kh, (((1,), (1,)), ((), ()))) * scale
        nk = nkv[0, h:h + 1, :]
        nv = nkv[1, h:h + 1, :]
        lnull = lax.dot_general(qh, nk, (((1,), (1,)), ((), ()))) * scale
        m = jnp.maximum(jnp.max(sim, axis=-1, keepdims=True), lnull)
        p = jnp.exp(sim - m)
        pn = jnp.exp(lnull - m)
        den = jnp.sum(p, axis=-1, keepdims=True) + pn
        oh = (lax.dot_general(p, vh, (((1,), (0,)), ((), ())))
              + pn * nv) / den
        outs.append(oh)
    o = jnp.concatenate(outs, axis=1)
    ao = lax.dot_general(o, wo_ref[0], (((1,), (0,)), ((), ())))
    out_ref[0, 0] = (ao - ntok_ref[0]) * (1.0 / NE)


def _experts(g4, grot4, lng, lnb, wq, wkv, wo, nkv, ntok):
    return pl.pallas_call(
        _k4_body,
        grid=(NE, B),
        in_specs=[
            pl.BlockSpec((1, 1, NRQ, D), lambda e, b: (b, e, 0, 0)),
            pl.BlockSpec((1, 1, NRKV, D), lambda e, b: (b, e + NE, 0, 0)),
            pl.BlockSpec((1, 1, NRQ, 2 * DH), lambda e, b: (b, e, 0, 0)),
            pl.BlockSpec((1, 1, NRKV, 2 * DH), lambda e, b: (b, e + NE, 0, 0)),
            pl.BlockSpec((1, 1, D), lambda e, b: (e, 0, 0)),
            pl.BlockSpec((1, 1, D), lambda e, b: (e, 0, 0)),
            pl.BlockSpec((1, D, INNER), lambda e, b: (e, 0, 0)),
            pl.BlockSpec((1, D, 2 * INNER), lambda e, b: (e, 0, 0)),
            pl.BlockSpec((1, INNER, D), lambda e, b: (e, 0, 0)),
            pl.BlockSpec((1, 2, H, DH), lambda e, b: (e, 0, 0, 0)),
            pl.BlockSpec((1, 1, D), lambda e, b: (e, 0, 0)),
        ],
        out_specs=pl.BlockSpec((1, 1, NRQ, D), lambda e, b: (e, b, 0, 0)),
        out_shape=jax.ShapeDtypeStruct((NE, B, NRQ, D), jnp.float32),
    )(g4, g4, grot4, grot4, lng, lnb, wq, wkv, wo, nkv, ntok)


def _k5_body(x_ref, d_ref, idx_ref, mn_ref, ffg_ref, ffb_ref,
             w1_ref, b1_ref, w2_ref, b2_ref, out_ref, *, tile):
    t = pl.program_id(1)
    xt = x_ref[0]
    rowid = (lax.broadcasted_iota(jnp.int32, (tile, 1), 0)
             + t * tile)
    acc = xt + mn_ref[...]
    for e in range(NE):
        ide = idx_ref[e:e + 1, :]
        me = jnp.where(ide == rowid, 1.0, 0.0)
        acc = acc + lax.dot_general(me, d_ref[e, 0],
                                    (((1,), (0,)), ((), ())), precision=HI)
    mu = jnp.mean(acc, axis=-1, keepdims=True)
    var = jnp.mean((acc - mu) ** 2, axis=-1, keepdims=True)
    h = (acc - mu) / jnp.sqrt(var + 1e-5) * ffg_ref[...] + ffb_ref[...]
    h = lax.dot_general(h, w1_ref[...], (((1,), (0,)), ((), ()))) + b1_ref[...]
    h = 0.5 * h * (1.0 + lax.erf(h * (2.0 ** -0.5)))
    h = lax.dot_general(h, w2_ref[...], (((1,), (0,)), ((), ()))) + b2_ref[...]
    out_ref[0] = h + acc


def _combine_ff(x3, deltas, idxl, mn, ffg, ffb, w1, b1, w2, b2):
    tile = 512
    body = functools.partial(_k5_body, tile=tile)
    return pl.pallas_call(
        body,
        grid=(B, SEQ // tile),
        in_specs=[
            pl.BlockSpec((1, tile, D), lambda b, t: (b, t, 0)),
            pl.BlockSpec((NE, 1, NRQ, D), lambda b, t: (0, b, 0, 0)),
            pl.BlockSpec((NROUTE, NRQ), lambda b, t: (b, 0)),
            pl.BlockSpec((1, D), lambda b, t: (0, 0)),
            pl.BlockSpec((1, D), lambda b, t: (0, 0)),
            pl.BlockSpec((1, D), lambda b, t: (0, 0)),
            pl.BlockSpec((D, D), lambda b, t: (0, 0)),
            pl.BlockSpec((1, D), lambda b, t: (0, 0)),
            pl.BlockSpec((D, D), lambda b, t: (0, 0)),
            pl.BlockSpec((1, D), lambda b, t: (0, 0)),
        ],
        out_specs=pl.BlockSpec((1, tile, D), lambda b, t: (b, t, 0)),
        out_shape=jax.ShapeDtypeStruct((B, SEQ, D), jnp.float32),
    )(x3, deltas, idxl, mn, ffg, ffb, w1, b1, w2, b2)


def _k6_body(x_ref, g_ref, b_ref, o_ref):
    xt = x_ref[...]
    mu = jnp.mean(xt, axis=-1, keepdims=True)
    var = jnp.mean((xt - mu) ** 2, axis=-1, keepdims=True)
    o_ref[...] = (xt - mu) / jnp.sqrt(var + 1e-5) * g_ref[...] + b_ref[...]


def _final_ln(x2d, g, b):
    return pl.pallas_call(
        _k6_body,
        grid=(B * SEQ // 512,),
        in_specs=[
            pl.BlockSpec((512, D), lambda t: (t, 0)),
            pl.BlockSpec((1, D), lambda t: (0, 0)),
            pl.BlockSpec((1, D), lambda t: (0, 0)),
        ],
        out_specs=pl.BlockSpec((512, D), lambda t: (t, 0)),
        out_shape=jax.ShapeDtypeStruct((B * SEQ, D), jnp.float32),
    )(x2d, g, b)


def kernel(x, rotary_emb, params):
    xc = x
    for layer in params['layers']:
        ex = layer['experts']
        routes = jnp.stack([e['q_route'] for e in ex]
                           + [e['kv_route'] for e in ex], axis=1)
        s = _scores(xc, routes)
        idxl, idxg = _select(s)
        rot128 = jnp.concatenate(
            [rotary_emb, jnp.zeros((SEQ, DH), jnp.float32)], axis=1)
        g, grot = _sc_gather(xc.reshape(B * SEQ, D), rot128,
                             idxg.reshape(-1), idxl.reshape(-1))
        g4 = g.reshape(B, NROUTE, NRQ, D)
        grot4 = grot.reshape(B, NROUTE, NRQ, 2 * DH)
        lng = jnp.stack([e['ln_g'] for e in ex])[:, None, :]
        lnb = jnp.stack([e['ln_b'] for e in ex])[:, None, :]
        wq = jnp.stack([e['Wq'] for e in ex])
        wkv = jnp.stack([e['Wkv'] for e in ex])
        wo = jnp.stack([e['Wo'] for e in ex])
        nkv = jnp.stack([e['null_kv'] for e in ex])
        ntok = jnp.stack([e['null_tokens'][0, 0] for e in ex])[:, None, :]
        deltas = _experts(g4, grot4, lng, lnb, wq, wkv, wo, nkv, ntok)
        mn = jnp.mean(ntok[:, 0, :], axis=0, keepdims=True)
        ff = layer['ff']
        xc = _combine_ff(
            xc, deltas, idxl, mn,
            ff['ln_g'][None, :], ff['ln_b'][None, :],
            ff['W1'], ff['b1'][None, :], ff['W2'], ff['b2'][None, :])
    out = _final_ln(xc.reshape(B * SEQ, D),
                    params['out_ln_g'][None, :], params['out_ln_b'][None, :])
    return out.reshape(B, SEQ, D)


# speedup vs baseline: 2.0838x; 1.0158x over previous
import functools

import jax
import jax.numpy as jnp
from jax import lax
from jax.experimental import pallas as pl
from jax.experimental.pallas import tpu as pltpu
from jax.experimental.pallas import tpu_sc as plsc

D = 1024
NL = 2
NE = 4
NRQ = 512
NRKV = 512
DH = 64
H = 4
INNER = H * DH
SEQ = 4096
B = 2
NROUTE = 2 * NE
NROWS = B * NROUTE
EFF_K = min(int(NRQ * 9 / 8), SEQ)
HI = jax.lax.Precision.HIGHEST


def _k1_body(x_ref, r_ref, s_ref):
    x = x_ref[0]
    r = r_ref[...]
    s_ref[...] = lax.dot_general(r, x, (((0,), (1,)), ((), ())),
                                 precision=jax.lax.Precision.DEFAULT)


def _scores(x, routes):
    return pl.pallas_call(
        _k1_body,
        grid=(B, SEQ // 512),
        in_specs=[
            pl.BlockSpec((1, 512, D), lambda b, t: (b, t, 0)),
            pl.BlockSpec((D, NROUTE), lambda b, t: (0, 0)),
        ],
        out_specs=pl.BlockSpec((NROUTE, 512), lambda b, t: (b, t)),
        out_shape=jax.ShapeDtypeStruct((NROWS, SEQ), jnp.float32),
    )(x, routes)


def _cumsum_lanes(x):
    n = x.shape[-1]
    k = 1
    while k < n:
        x = x + jnp.concatenate(
            [jnp.zeros(x.shape[:-1] + (k,), x.dtype), x[..., :-k]], axis=-1)
        k *= 2
    return x


def _k2_body(s_ref, idxl_ref, idxg_ref):
    s = s_ref[...]
    logk = jnp.log(jnp.float32(EFF_K))
    b = -s
    a = jnp.zeros((NROWS, 1), jnp.float32)
    cur = 4.0
    for _ in range(20):
        sb = (s + b) / cur
        m = jnp.max(sb, axis=-1, keepdims=True)
        lse = jnp.log(jnp.sum(jnp.exp(sb - m), axis=-1, keepdims=True)) + m
        a = cur * (logk - lse)
        b = -jnp.maximum(s + a, 0.0)
        cur = max(cur * 0.7, 0.03)
    key = jnp.minimum(s + a, 0.0)
    scr = jnp.exp(key / cur)
    ki = lax.bitcast_convert_type(scr, jnp.int32)

    lo = jnp.zeros((NROWS, 1), jnp.int32)
    hi = jnp.full((NROWS, 1), 1065353216, jnp.int32)
    for _ in range(31):
        mid = lo + lax.shift_right_arithmetic(hi - lo + 1, 1)
        cnt = jnp.sum((ki >= mid).astype(jnp.float32), axis=-1, keepdims=True)
        pred = cnt >= float(NRQ)
        lo = jnp.where(pred, mid, lo)
        hi = jnp.where(pred, hi, mid - 1)
    T = lo

    gt = ki > T
    eq = ki == T
    c_gt = jnp.sum(gt.astype(jnp.float32), axis=-1, keepdims=True)
    need = float(NRQ) - c_gt
    eqf = eq.astype(jnp.float32)
    excl_eq = _cumsum_lanes(eqf) - eqf
    mask = jnp.logical_or(gt, jnp.logical_and(eq, excl_eq < need))
    maskf = mask.astype(jnp.float32)
    slot = _cumsum_lanes(maskf) - maskf

    jj = lax.broadcasted_iota(jnp.int32, (NRQ, SEQ), 0).astype(jnp.float32)
    iv = lax.broadcasted_iota(jnp.int32, (1, SEQ), 1).astype(jnp.float32)
    rows = []
    for r in range(NROWS):
        srow = slot[r:r + 1, :]
        mrow = maskf[r:r + 1, :]
        e = jnp.where(jnp.logical_and(srow == jj, mrow > 0.5), 1.0, 0.0)
        rows.append(lax.dot_general(iv, e, (((1,), (1,)), ((), ())),
                                    precision=HI))
    idxf = jnp.concatenate(rows, axis=0)
    idxl = idxf.astype(jnp.int32)
    roff = jnp.where(
        lax.broadcasted_iota(jnp.int32, (NROWS, 1), 0) >= NROUTE, SEQ, 0)
    idxl_ref[...] = idxl
    idxg_ref[...] = idxl + roff


def _select(s):
    return pl.pallas_call(
        _k2_body,
        in_specs=[pl.BlockSpec((NROWS, SEQ), lambda: (0, 0))],
        out_specs=[
            pl.BlockSpec((NROWS, NRQ), lambda: (0, 0)),
            pl.BlockSpec((NROWS, NRQ), lambda: (0, 0)),
        ],
        out_shape=[
            jax.ShapeDtypeStruct((NROWS, NRQ), jnp.int32),
            jax.ShapeDtypeStruct((NROWS, NRQ), jnp.int32),
        ],
    )(s)


_NW = 32
_RPW = (NROWS * NRQ) // _NW
_CH = 64
_NCH = _RPW // _CH


def _sc_gather_body(x_hbm, rot_hbm, gidx_hbm, lidx_hbm, g_hbm, grot_hbm,
                    idxg_v, idxl_v, rows_v, rrows_v, sem):
    wid = lax.axis_index("s") * 2 + lax.axis_index("c")
    pltpu.sync_copy(gidx_hbm.at[wid], idxg_v)
    pltpu.sync_copy(lidx_hbm.at[wid], idxl_v)
    base = wid * _RPW
    for c in range(_NCH):
        pltpu.async_copy(x_hbm.at[idxg_v.at[c]], rows_v, sem).wait()
        pltpu.sync_copy(rows_v, g_hbm.at[pl.ds(base + c * _CH, _CH)])
        pltpu.async_copy(rot_hbm.at[idxl_v.at[c]], rrows_v, sem).wait()
        pltpu.sync_copy(rrows_v, grot_hbm.at[pl.ds(base + c * _CH, _CH)])


def _sc_gather(x2d, rot, idxg, idxl):
    mesh = plsc.VectorSubcoreMesh(core_axis_name="c", subcore_axis_name="s")
    fn = functools.partial(
        pl.kernel, mesh=mesh,
        out_type=[
            jax.ShapeDtypeStruct((NROWS * NRQ, D), jnp.float32),
            jax.ShapeDtypeStruct((NROWS * NRQ, 2 * DH), jnp.float32),
        ],
        scratch_types=[
            pltpu.VMEM((_NCH, _CH), jnp.int32),
            pltpu.VMEM((_NCH, _CH), jnp.int32),
            pltpu.VMEM((_CH, D), jnp.float32),
            pltpu.VMEM((_CH, 2 * DH), jnp.float32),
            pltpu.SemaphoreType.DMA,
        ],
    )(_sc_gather_body)
    return fn(x2d, rot,
              idxg.reshape(_NW, _NCH, _CH), idxl.reshape(_NW, _NCH, _CH))


def _k4_body(gq_ref, gkv_ref, rq_ref, rkv_ref, lng_ref, lnb_ref,
             wq_ref, wkv_ref, wo_ref, nkv_ref, ntok_ref, out_ref):
    gq = gq_ref[0, 0]
    gkv = gkv_ref[0, 0]
    rq = rq_ref[0, 0, :, :DH]
    rkv = rkv_ref[0, 0, :, :DH]
    g = lng_ref[0]
    bb = lnb_ref[0]

    def ln(t):
        mu = jnp.mean(t, axis=-1, keepdims=True)
        var = jnp.mean((t - mu) ** 2, axis=-1, keepdims=True)
        return (t - mu) / jnp.sqrt(var + 1e-5) * g + bb

    xq = ln(gq)
    ctx = ln(gkv)
    q = lax.dot_general(xq, wq_ref[0], (((1,), (0,)), ((), ())))
    kv = lax.dot_general(ctx, wkv_ref[0], (((1,), (0,)), ((), ())))
    k = kv[:, :INNER]
    v = kv[:, INNER:]

    cq, sq = jnp.cos(rq), jnp.sin(rq)
    ck, sk = jnp.cos(rkv), jnp.sin(rkv)
    cq4 = jnp.concatenate([cq] * H, axis=1)
    sq4 = jnp.concatenate([sq] * H, axis=1)
    ck4 = jnp.concatenate([ck] * H, axis=1)
    sk4 = jnp.concatenate([sk] * H, axis=1)

    def rot_half(t):
        parts = []
        for h in range(H):
            t1 = t[:, h * DH:h * DH + DH // 2]
            t2 = t[:, h * DH + DH // 2:(h + 1) * DH]
            parts.append(jnp.concatenate([-t2, t1], axis=1))
        return jnp.concatenate(parts, axis=1)

    q = q * cq4 + rot_half(q) * sq4
    k = k * ck4 + rot_half(k) * sk4

    nkv = nkv_ref[0]
    scale = DH ** -0.5
    outs = []
    for h in range(H):
        qh = q[:, h * DH:(h + 1) * DH]
        kh = k[:, h * DH:(h + 1) * DH]
        vh = v[:, h * DH:(h + 1) * DH]
        sim = lax.dot_general(qh, kh, (((1,), (1,)), ((), ()))) * scale
        nk = nkv[0, h:h + 1, :]
        nv = nkv[1, h:h + 1, :]
        lnull = lax.dot_general(qh, nk, (((1,), (1,)), ((), ()))) * scale
        m = jnp.maximum(jnp.max(sim, axis=-1, keepdims=True), lnull)
        p = jnp.exp(sim - m)
        pn = jnp.exp(lnull - m)
        den = jnp.sum(p, axis=-1, keepdims=True) + pn
        oh = (lax.dot_general(p, vh, (((1,), (0,)), ((), ())))
              + pn * nv) / den
        outs.append(oh)
    o = jnp.concatenate(outs, axis=1)
    ao = lax.dot_general(o, wo_ref[0], (((1,), (0,)), ((), ())))
    out_ref[0, 0] = (ao - ntok_ref[0]) * (1.0 / NE)


def _experts(g4, grot4, lng, lnb, wq, wkv, wo, nkv, ntok):
    return pl.pallas_call(
        _k4_body,
        grid=(NE, B),
        in_specs=[
            pl.BlockSpec((1, 1, NRQ, D), lambda e, b: (b, e, 0, 0)),
            pl.BlockSpec((1, 1, NRKV, D), lambda e, b: (b, e + NE, 0, 0)),
            pl.BlockSpec((1, 1, NRQ, 2 * DH), lambda e, b: (b, e, 0, 0)),
            pl.BlockSpec((1, 1, NRKV, 2 * DH), lambda e, b: (b, e + NE, 0, 0)),
            pl.BlockSpec((1, 1, D), lambda e, b: (e, 0, 0)),
            pl.BlockSpec((1, 1, D), lambda e, b: (e, 0, 0)),
            pl.BlockSpec((1, D, INNER), lambda e, b: (e, 0, 0)),
            pl.BlockSpec((1, D, 2 * INNER), lambda e, b: (e, 0, 0)),
            pl.BlockSpec((1, INNER, D), lambda e, b: (e, 0, 0)),
            pl.BlockSpec((1, 2, H, DH), lambda e, b: (e, 0, 0, 0)),
            pl.BlockSpec((1, 1, D), lambda e, b: (e, 0, 0)),
        ],
        out_specs=pl.BlockSpec((1, 1, NRQ, D), lambda e, b: (e, b, 0, 0)),
        out_shape=jax.ShapeDtypeStruct((NE, B, NRQ, D), jnp.float32),
    )(g4, g4, grot4, grot4, lng, lnb, wq, wkv, wo, nkv, ntok)


def _k5_body(x_ref, d_ref, idx_ref, mn_ref, ffg_ref, ffb_ref,
             w1_ref, b1_ref, w2_ref, b2_ref, og_ref, ob_ref, out_ref, *,
             tile, final_ln):
    t = pl.program_id(1)
    xt = x_ref[0]
    rowid = (lax.broadcasted_iota(jnp.int32, (tile, 1), 0)
             + t * tile)
    acc = xt + mn_ref[...]
    for e in range(NE):
        ide = idx_ref[e:e + 1, :]
        me = jnp.where(ide == rowid, 1.0, 0.0)
        acc = acc + lax.dot_general(me, d_ref[e, 0],
                                    (((1,), (0,)), ((), ())), precision=HI)
    mu = jnp.mean(acc, axis=-1, keepdims=True)
    var = jnp.mean((acc - mu) ** 2, axis=-1, keepdims=True)
    h = (acc - mu) / jnp.sqrt(var + 1e-5) * ffg_ref[...] + ffb_ref[...]
    h = lax.dot_general(h, w1_ref[...], (((1,), (0,)), ((), ()))) + b1_ref[...]
    h = 0.5 * h * (1.0 + lax.erf(h * (2.0 ** -0.5)))
    h = lax.dot_general(h, w2_ref[...], (((1,), (0,)), ((), ()))) + b2_ref[...]
    y = h + acc
    if final_ln:
        mu2 = jnp.mean(y, axis=-1, keepdims=True)
        var2 = jnp.mean((y - mu2) ** 2, axis=-1, keepdims=True)
        y = (y - mu2) / jnp.sqrt(var2 + 1e-5) * og_ref[...] + ob_ref[...]
    out_ref[0] = y


def _combine_ff(x3, deltas, idxl, mn, ffg, ffb, w1, b1, w2, b2, og, ob,
                final_ln):
    tile = 512
    body = functools.partial(_k5_body, tile=tile, final_ln=final_ln)
    return pl.pallas_call(
        body,
        grid=(B, SEQ // tile),
        in_specs=[
            pl.BlockSpec((1, tile, D), lambda b, t: (b, t, 0)),
            pl.BlockSpec((NE, 1, NRQ, D), lambda b, t: (0, b, 0, 0)),
            pl.BlockSpec((NROUTE, NRQ), lambda b, t: (b, 0)),
            pl.BlockSpec((1, D), lambda b, t: (0, 0)),
            pl.BlockSpec((1, D), lambda b, t: (0, 0)),
            pl.BlockSpec((1, D), lambda b, t: (0, 0)),
            pl.BlockSpec((D, D), lambda b, t: (0, 0)),
            pl.BlockSpec((1, D), lambda b, t: (0, 0)),
            pl.BlockSpec((D, D), lambda b, t: (0, 0)),
            pl.BlockSpec((1, D), lambda b, t: (0, 0)),
            pl.BlockSpec((1, D), lambda b, t: (0, 0)),
            pl.BlockSpec((1, D), lambda b, t: (0, 0)),
        ],
        out_specs=pl.BlockSpec((1, tile, D), lambda b, t: (b, t, 0)),
        out_shape=jax.ShapeDtypeStruct((B, SEQ, D), jnp.float32),
    )(x3, deltas, idxl, mn, ffg, ffb, w1, b1, w2, b2, og, ob)


def _k6_body(x_ref, g_ref, b_ref, o_ref):
    xt = x_ref[...]
    mu = jnp.mean(xt, axis=-1, keepdims=True)
    var = jnp.mean((xt - mu) ** 2, axis=-1, keepdims=True)
    o_ref[...] = (xt - mu) / jnp.sqrt(var + 1e-5) * g_ref[...] + b_ref[...]


def _final_ln(x2d, g, b):
    return pl.pallas_call(
        _k6_body,
        grid=(B * SEQ // 512,),
        in_specs=[
            pl.BlockSpec((512, D), lambda t: (t, 0)),
            pl.BlockSpec((1, D), lambda t: (0, 0)),
            pl.BlockSpec((1, D), lambda t: (0, 0)),
        ],
        out_specs=pl.BlockSpec((512, D), lambda t: (t, 0)),
        out_shape=jax.ShapeDtypeStruct((B * SEQ, D), jnp.float32),
    )(x2d, g, b)


def kernel(x, rotary_emb, params):
    xc = x
    og = params['out_ln_g'][None, :]
    ob = params['out_ln_b'][None, :]
    nlayers = len(params['layers'])
    for li, layer in enumerate(params['layers']):
        ex = layer['experts']
        routes = jnp.stack([e['q_route'] for e in ex]
                           + [e['kv_route'] for e in ex], axis=1)
        s = _scores(xc, routes)
        idxl, idxg = _select(s)
        rot128 = jnp.concatenate(
            [rotary_emb, jnp.zeros((SEQ, DH), jnp.float32)], axis=1)
        g, grot = _sc_gather(xc.reshape(B * SEQ, D), rot128,
                             idxg.reshape(-1), idxl.reshape(-1))
        g4 = g.reshape(B, NROUTE, NRQ, D)
        grot4 = grot.reshape(B, NROUTE, NRQ, 2 * DH)
        lng = jnp.stack([e['ln_g'] for e in ex])[:, None, :]
        lnb = jnp.stack([e['ln_b'] for e in ex])[:, None, :]
        wq = jnp.stack([e['Wq'] for e in ex])
        wkv = jnp.stack([e['Wkv'] for e in ex])
        wo = jnp.stack([e['Wo'] for e in ex])
        nkv = jnp.stack([e['null_kv'] for e in ex])
        ntok = jnp.stack([e['null_tokens'][0, 0] for e in ex])[:, None, :]
        deltas = _experts(g4, grot4, lng, lnb, wq, wkv, wo, nkv, ntok)
        mn = jnp.mean(ntok[:, 0, :], axis=0, keepdims=True)
        ff = layer['ff']
        xc = _combine_ff(
            xc, deltas, idxl, mn,
            ff['ln_g'][None, :], ff['ln_b'][None, :],
            ff['W1'], ff['b1'][None, :], ff['W2'], ff['b2'][None, :],
            og, ob, final_ln=(li == nlayers - 1))
    return xc
